# baseline (speedup 1.0000x reference)
import jax
import jax.numpy as jnp
from jax import lax
from jax.experimental import pallas as pl
from jax.experimental.pallas import tpu as pltpu
from jax.experimental.pallas import tpu_sc as plsc

B, S = 4096, 200
BS = B * S

VOCABS = [7, 7, 2, 3, 4, 2, 10, 3]
WIDTHS = [16, 16, 8, 8, 16, 8, 16, 16]
WOFFS = [0, 16, 32, 40, 48, 64, 72, 88]
NA, NB_ = 294, 240
STRIDES = [42 * NB_, 6 * NB_, 3 * NB_, NB_, 60, 30, 3, 1]
ADIGS = [(42, 7), (6, 7), (3, 2), (1, 3)]
BDIGS = [(60, 4), (30, 2), (3, 10), (1, 3)]

QA_ROWS_PER_STEP = 6


def _table_body(e0, e1, e2, e3, e4, e5, e6, e7, w_ref, b_ref, out_ref,
                qa_ref, qb_ref):
    es = [e0, e1, e2, e3, e4, e5, e6, e7]

    @pl.when(pl.program_id(0) == 0)
    def _build_quads():
        ps = []
        for t in range(8):
            ps.append(jnp.dot(es[t][...], w_ref[WOFFS[t]:WOFFS[t] + WIDTHS[t], :],
                              preferred_element_type=jnp.float32))
        qa = jnp.broadcast_to(b_ref[...], (NA, 128))
        for t, (stride, voc) in enumerate(ADIGS):
            r = lax.broadcasted_iota(jnp.int32, (NA, VOCABS[t]), 0)
            c = lax.broadcasted_iota(jnp.int32, (NA, VOCABS[t]), 1)
            sel = ((r // stride) % voc == c).astype(jnp.float32)
            qa = qa + jnp.dot(sel, ps[t], preferred_element_type=jnp.float32)
        qa_ref[...] = qa
        qb = jnp.zeros((NB_, 128), jnp.float32)
        for k, (stride, voc) in enumerate(BDIGS):
            t = 4 + k
            r = lax.broadcasted_iota(jnp.int32, (NB_, VOCABS[t]), 0)
            c = lax.broadcasted_iota(jnp.int32, (NB_, VOCABS[t]), 1)
            sel = ((r // stride) % voc == c).astype(jnp.float32)
            qb = qb + jnp.dot(sel, ps[t], preferred_element_type=jnp.float32)
        qb_ref[...] = qb

    i = pl.program_id(0)
    qa_rows = qa_ref[pl.ds(i * QA_ROWS_PER_STEP, QA_ROWS_PER_STEP), :]
    out_ref[...] = qa_rows[:, None, :] + qb_ref[...][None, :, :]


def _build_table(es, W, b2):
    in_specs = []
    for t in range(8):
        in_specs.append(pl.BlockSpec((VOCABS[t], WIDTHS[t]), lambda i: (0, 0)))
    in_specs.append(pl.BlockSpec((104, 128), lambda i: (0, 0)))
    in_specs.append(pl.BlockSpec((1, 128), lambda i: (0, 0)))
    return pl.pallas_call(
        _table_body,
        grid=(NA // QA_ROWS_PER_STEP,),
        in_specs=in_specs,
        out_specs=pl.BlockSpec((QA_ROWS_PER_STEP, NB_, 128), lambda i: (i, 0, 0)),
        out_shape=jax.ShapeDtypeStruct((NA, NB_, 128), jnp.float32),
        scratch_shapes=[pltpu.VMEM((NA, 128), jnp.float32),
                        pltpu.VMEM((NB_, 128), jnp.float32)],
        compiler_params=pltpu.CompilerParams(
            dimension_semantics=("arbitrary",)),
    )(*es, W, b2)


IDX_TR = 2048


def _idx_body(tok_ref, out_ref):
    tokf = tok_ref[...].astype(jnp.float32)
    l = lax.broadcasted_iota(jnp.int32, (128, 16), 0)
    g = lax.broadcasted_iota(jnp.int32, (128, 16), 1)
    seg = (l // 8 == g)
    sl = l % 8
    idxf = jnp.zeros((IDX_TR, 16), jnp.float32)
    for mask in (~0x3F, 0x3F):
        sv = jnp.zeros((128, 16), jnp.float32)
        for t in range(8):
            part = float(STRIDES[t] & mask)
            sv = jnp.where(seg & (sl == t), part, sv)
        idxf = idxf + jnp.dot(tokf, sv, preferred_element_type=jnp.float32)
    out_ref[...] = (idxf + 0.5).astype(jnp.int32)


def _build_idx(tok16):
    return pl.pallas_call(
        _idx_body,
        grid=(BS // 16 // IDX_TR,),
        in_specs=[pl.BlockSpec((IDX_TR, 128), lambda i: (i, 0))],
        out_specs=pl.BlockSpec((IDX_TR, 16), lambda i: (i, 0)),
        out_shape=jax.ShapeDtypeStruct((BS // 16, 16), jnp.int32),
    )(tok16)


NC, NS = 2, 16
NW = NC * NS
CH = 128
CHUNKS = 5
OUTER = CH * CHUNKS
TOK_PER_W = BS // NW
NOUT = TOK_PER_W // OUTER


GTOK = 400
GSUB = 5
GROWS = GTOK // GSUB
NG = TOK_PER_W // GTOK


def _sc_body(bt_hbm, idx_hbm, out_hbm, idx_v, buf_a, buf_b,
             sg_a, sg_b, so_a, so_b):
    bufs = [buf_a, buf_b]
    sg = [sg_a, sg_b]
    so = [so_a, so_b]
    wid = lax.axis_index("s") * NC + lax.axis_index("c")
    tok0 = wid * TOK_PER_W

    def gather_start(g, p):
        for j in range(GSUB):
            pltpu.make_async_copy(
                bt_hbm.at[idx_v.at[pl.ds(g * GTOK + j * GROWS, GROWS)]],
                bufs[p].at[pl.ds(j * GROWS, GROWS), :], sg[p]).start()

    def gather_wait(g, p):
        for j in range(GSUB):
            pltpu.make_async_copy(
                bt_hbm.at[idx_v.at[pl.ds(g * GTOK + j * GROWS, GROWS)]],
                bufs[p].at[pl.ds(j * GROWS, GROWS), :], sg[p]).wait()

    class _G:
        def __init__(self, g, p):
            self.g, self.p = g, p

        def start(self):
            gather_start(self.g, self.p)

        def wait(self):
            gather_wait(self.g, self.p)

    def gather_copy(g, p):
        return _G(g, p)

    def out_copy(g, p):
        return pltpu.make_async_copy(
            bufs[p], out_hbm.at[pl.ds(tok0 + g * GTOK, GTOK), :], so[p])

    pltpu.sync_copy(idx_hbm.at[pl.ds(tok0, TOK_PER_W)], idx_v)

    gather_copy(0, 0).start()

    def grp_body(i, carry):
        for p in range(2):
            g = i * 2 + p
            if p == 0:
                @pl.when(i > 0)
                def _other_free():
                    out_copy(g - 1, 1).wait()
                gather_copy(g + 1, 1).start()
            else:
                out_copy(g - 1, 0).wait()

                @pl.when(i < NG // 2 - 1)
                def _fire_next():
                    gather_copy(g + 1, 0).start()
            gather_copy(g, p).wait()
            out_copy(g, p).start()
        return carry

    lax.fori_loop(0, NG // 2, grp_body, 0)
    out_copy(NG - 1, 1).wait()


def _sc_gather(bt, idx1):
    mesh = plsc.VectorSubcoreMesh(core_axis_name="c", subcore_axis_name="s",
                                  num_cores=NC, num_subcores=NS)
    return pl.kernel(
        _sc_body,
        out_type=jax.ShapeDtypeStruct((BS, 128), jnp.float32),
        mesh=mesh,
        scratch_types=[pltpu.VMEM((TOK_PER_W,), jnp.int32),
                       pltpu.VMEM((GTOK, 128), jnp.float32),
                       pltpu.VMEM((GTOK, 128), jnp.float32),
                       pltpu.SemaphoreType.DMA,
                       pltpu.SemaphoreType.DMA,
                       pltpu.SemaphoreType.DMA,
                       pltpu.SemaphoreType.DMA],
    )(bt, idx1)


def kernel(tokens, E0, E1, E2, E3, E4, E5, E6, E7, W, b):
    es = [E0, E1, E2, E3, E4, E5, E6, E7]
    bt = _build_table(es, W, b.reshape(1, 128)).reshape(NA * NB_, 128)
    idx = _build_idx(tokens.reshape(BS // 16, 128)).reshape(BS)
    out = _sc_gather(bt, idx)
    return out.reshape(B, S, 128)


# speedup vs baseline: 1.3948x; 1.3948x over previous
import jax
import jax.numpy as jnp
from jax import lax
from jax.experimental import pallas as pl
from jax.experimental.pallas import tpu as pltpu
from jax.experimental.pallas import tpu_sc as plsc

B, S = 4096, 200
BS = B * S

VOCABS = [7, 7, 2, 3, 4, 2, 10, 3]
WIDTHS = [16, 16, 8, 8, 16, 8, 16, 16]
WOFFS = [0, 16, 32, 40, 48, 64, 72, 88]
NA, NB_ = 294, 240
STRIDES = [42 * NB_, 6 * NB_, 3 * NB_, NB_, 60, 30, 3, 1]
ADIGS = [(42, 7), (6, 7), (3, 2), (1, 3)]
BDIGS = [(60, 4), (30, 2), (3, 10), (1, 3)]

QA_ROWS_PER_STEP = 6


def _table_body(e0, e1, e2, e3, e4, e5, e6, e7, w_ref, b_ref, out_ref,
                qa_ref, qb_ref):
    es = [e0, e1, e2, e3, e4, e5, e6, e7]

    @pl.when(pl.program_id(0) == 0)
    def _build_quads():
        ps = []
        for t in range(8):
            ps.append(jnp.dot(es[t][...], w_ref[WOFFS[t]:WOFFS[t] + WIDTHS[t], :],
                              preferred_element_type=jnp.float32))
        qa = jnp.broadcast_to(b_ref[...], (NA, 128))
        for t, (stride, voc) in enumerate(ADIGS):
            r = lax.broadcasted_iota(jnp.int32, (NA, VOCABS[t]), 0)
            c = lax.broadcasted_iota(jnp.int32, (NA, VOCABS[t]), 1)
            sel = ((r // stride) % voc == c).astype(jnp.float32)
            qa = qa + jnp.dot(sel, ps[t], preferred_element_type=jnp.float32)
        qa_ref[...] = qa
        qb = jnp.zeros((NB_, 128), jnp.float32)
        for k, (stride, voc) in enumerate(BDIGS):
            t = 4 + k
            r = lax.broadcasted_iota(jnp.int32, (NB_, VOCABS[t]), 0)
            c = lax.broadcasted_iota(jnp.int32, (NB_, VOCABS[t]), 1)
            sel = ((r // stride) % voc == c).astype(jnp.float32)
            qb = qb + jnp.dot(sel, ps[t], preferred_element_type=jnp.float32)
        qb_ref[...] = qb

    i = pl.program_id(0)
    qa_rows = qa_ref[pl.ds(i * QA_ROWS_PER_STEP, QA_ROWS_PER_STEP), :]
    out_ref[...] = qa_rows[:, None, :] + qb_ref[...][None, :, :]


def _build_table(es, W, b2):
    in_specs = []
    for t in range(8):
        in_specs.append(pl.BlockSpec((VOCABS[t], WIDTHS[t]), lambda i: (0, 0)))
    in_specs.append(pl.BlockSpec((104, 128), lambda i: (0, 0)))
    in_specs.append(pl.BlockSpec((1, 128), lambda i: (0, 0)))
    return pl.pallas_call(
        _table_body,
        grid=(NA // QA_ROWS_PER_STEP,),
        in_specs=in_specs,
        out_specs=pl.BlockSpec((QA_ROWS_PER_STEP, NB_, 128), lambda i: (i, 0, 0)),
        out_shape=jax.ShapeDtypeStruct((NA, NB_, 128), jnp.float32),
        scratch_shapes=[pltpu.VMEM((NA, 128), jnp.float32),
                        pltpu.VMEM((NB_, 128), jnp.float32)],
        compiler_params=pltpu.CompilerParams(
            dimension_semantics=("arbitrary",)),
    )(*es, W, b2)


IDX_TR = 1024


def _idx_body(tok_ref, out_ref):
    tokf = tok_ref[...].astype(jnp.float32)
    l = lax.broadcasted_iota(jnp.int32, (1600, 200), 0)
    t2 = lax.broadcasted_iota(jnp.int32, (1600, 200), 1)
    seg = (l // 8 == t2)
    sl = l % 8
    idxf = jnp.zeros((IDX_TR, 200), jnp.float32)
    for mask in (~0x3F, 0x3F):
        sv = jnp.zeros((1600, 200), jnp.float32)
        for t in range(8):
            part = float(STRIDES[t] & mask)
            sv = jnp.where(seg & (sl == t), part, sv)
        idxf = idxf + jnp.dot(tokf, sv, preferred_element_type=jnp.float32)
    out_ref[...] = (idxf + 0.5).astype(jnp.int32)


def _build_idx(tok16):
    return pl.pallas_call(
        _idx_body,
        grid=(B // IDX_TR,),
        in_specs=[pl.BlockSpec((IDX_TR, 1600), lambda i: (i, 0))],
        out_specs=pl.BlockSpec((IDX_TR, 200), lambda i: (i, 0)),
        out_shape=jax.ShapeDtypeStruct((B, 200), jnp.int32),
    )(tok16)


NC, NS = 2, 16
NW = NC * NS
CH = 128
CHUNKS = 5
OUTER = CH * CHUNKS
TOK_PER_W = BS // NW
NOUT = TOK_PER_W // OUTER


GTOK = 400
GSUB = 5
GROWS = GTOK // GSUB
NG = TOK_PER_W // GTOK


def _sc_body(bt_hbm, idx_hbm, out_hbm, idx_v, buf_a, buf_b,
             sg_a, sg_b, so_a, so_b):
    bufs = [buf_a, buf_b]
    sg = [sg_a, sg_b]
    so = [so_a, so_b]
    wid = lax.axis_index("s") * NC + lax.axis_index("c")
    tok0 = wid * TOK_PER_W

    def gather_start(g, p):
        for j in range(GSUB):
            pltpu.make_async_copy(
                bt_hbm.at[idx_v.at[pl.ds(g * GTOK + j * GROWS, GROWS)]],
                bufs[p].at[pl.ds(j * GROWS, GROWS), :], sg[p]).start()

    def gather_wait(g, p):
        for j in range(GSUB):
            pltpu.make_async_copy(
                bt_hbm.at[idx_v.at[pl.ds(g * GTOK + j * GROWS, GROWS)]],
                bufs[p].at[pl.ds(j * GROWS, GROWS), :], sg[p]).wait()

    class _G:
        def __init__(self, g, p):
            self.g, self.p = g, p

        def start(self):
            gather_start(self.g, self.p)

        def wait(self):
            gather_wait(self.g, self.p)

    def gather_copy(g, p):
        return _G(g, p)

    def out_copy(g, p):
        return pltpu.make_async_copy(
            bufs[p], out_hbm.at[pl.ds(tok0 + g * GTOK, GTOK), :], so[p])

    pltpu.sync_copy(idx_hbm.at[pl.ds(tok0, TOK_PER_W)], idx_v)

    gather_copy(0, 0).start()

    def grp_body(i, carry):
        for p in range(2):
            g = i * 2 + p
            if p == 0:
                @pl.when(i > 0)
                def _other_free():
                    out_copy(g - 1, 1).wait()
                gather_copy(g + 1, 1).start()
            else:
                out_copy(g - 1, 0).wait()

                @pl.when(i < NG // 2 - 1)
                def _fire_next():
                    gather_copy(g + 1, 0).start()
            gather_copy(g, p).wait()
            out_copy(g, p).start()
        return carry

    lax.fori_loop(0, NG // 2, grp_body, 0)
    out_copy(NG - 1, 1).wait()


def _sc_gather(bt, idx1):
    mesh = plsc.VectorSubcoreMesh(core_axis_name="c", subcore_axis_name="s",
                                  num_cores=NC, num_subcores=NS)
    return pl.kernel(
        _sc_body,
        out_type=jax.ShapeDtypeStruct((BS, 128), jnp.float32),
        mesh=mesh,
        scratch_types=[pltpu.VMEM((TOK_PER_W,), jnp.int32),
                       pltpu.VMEM((GTOK, 128), jnp.float32),
                       pltpu.VMEM((GTOK, 128), jnp.float32),
                       pltpu.SemaphoreType.DMA,
                       pltpu.SemaphoreType.DMA,
                       pltpu.SemaphoreType.DMA,
                       pltpu.SemaphoreType.DMA],
    )(bt, idx1)


def kernel(tokens, E0, E1, E2, E3, E4, E5, E6, E7, W, b):
    es = [E0, E1, E2, E3, E4, E5, E6, E7]
    bt = _build_table(es, W, b.reshape(1, 128)).reshape(NA * NB_, 128)
    idx = _build_idx(tokens.reshape(B, S * 8)).reshape(BS)
    out = _sc_gather(bt, idx)
    return out.reshape(B, S, 128)
